# baseline (speedup 1.0000x reference)
import functools

import jax
import jax.numpy as jnp
from jax import lax
from jax.experimental import pallas as pl
from jax.experimental.pallas import tpu as pltpu
from jax.experimental.pallas import tpu_sc as plsc

T, H, I, E, K = 2048, 2048, 1024, 8, 2
TK = T * K
TM = 128
NUM_TILES = TK // TM
W_ITEMS = NUM_TILES + E - 1
TT = 256
NT = T // TT

NC, NS, L = 2, 16, 16
NW = NC * NS
TPW = T // NW
CH = 16
def _sc_mesh():
    return plsc.VectorSubcoreMesh(core_axis_name="c", subcore_axis_name="s",
                                  num_cores=NC, num_subcores=NS)


_GDN = jax.lax.GatherDimensionNumbers(
    offset_dims=(), collapsed_slice_dims=(0,), start_index_map=(0,))


def _splat(v, idx):
    return jax.lax.gather(v, idx[:, None], _GDN, (1,),
                          mode=jax.lax.GatherScatterMode.PROMISE_IN_BOUNDS)


def _dotT(a, b):
    return jax.lax.dot_general(a, b, (((1,), (1,)), ((), ())),
                               preferred_element_type=jnp.float32,
                               precision=jax.lax.Precision.DEFAULT)


def _shared_router_body(x_ref, gw_ref, sg_ref, su_ref, sd_ref,
                        out_ref, idx_ref, w_ref):
    xt = x_ref[...]
    g = _dotT(xt, sg_ref[...])
    u = _dotT(xt, su_ref[...])
    h = g * jax.nn.sigmoid(g) * u
    out_ref[...] = _dotT(h, sd_ref[...])

    logitsT = jax.lax.dot_general(gw_ref[...], xt, (((1,), (1,)), ((), ())),
                                  preferred_element_type=jnp.float32,
                                  precision=jax.lax.Precision.DEFAULT)
    iota = jax.lax.broadcasted_iota(jnp.int32, (E, TT), 0)
    l1 = jnp.max(logitsT, axis=0, keepdims=True)
    i1 = jnp.min(jnp.where(logitsT == l1, iota, E), axis=0, keepdims=True)
    l2m = jnp.where(iota == i1, -jnp.inf, logitsT)
    l2 = jnp.max(l2m, axis=0, keepdims=True)
    i2 = jnp.min(jnp.where(l2m == l2, iota, E), axis=0, keepdims=True)
    r = jnp.exp(l2 - l1)
    w1v = 1.0 / (1.0 + r)
    idx_ref[...] = jnp.concatenate([i1, i2], axis=0)
    w_ref[...] = jnp.concatenate([w1v, 1.0 - w1v], axis=0)


def _shared_router(x_flat, gate_w, sg, su, sd):
    return pl.pallas_call(
        _shared_router_body,
        grid=(NT,),
        in_specs=[
            pl.BlockSpec((TT, H), lambda i: (i, 0)),
            pl.BlockSpec((E, H), lambda i: (0, 0)),
            pl.BlockSpec((I, H), lambda i: (0, 0)),
            pl.BlockSpec((I, H), lambda i: (0, 0)),
            pl.BlockSpec((H, I), lambda i: (0, 0)),
        ],
        out_specs=[
            pl.BlockSpec((TT, H), lambda i: (i, 0)),
            pl.BlockSpec((K, TT), lambda i: (0, i)),
            pl.BlockSpec((K, TT), lambda i: (0, i)),
        ],
        out_shape=[
            jax.ShapeDtypeStruct((T, H), jnp.float32),
            jax.ShapeDtypeStruct((K, T), jnp.int32),
            jax.ShapeDtypeStruct((K, T), jnp.float32),
        ],
    )(x_flat, gate_w, sg, su, sd)


def _sort_body(e_ref, pos_ref, eid_ref, mt_ref, lo_ref, hi_ref, first_ref,
               off_ref):
    ev = e_ref[...]
    r128 = jax.lax.broadcasted_iota(jnp.int32, (128, 128), 0)
    c128 = jax.lax.broadcasted_iota(jnp.int32, (128, 128), 1)
    incl_mat = (r128 <= c128).astype(jnp.float32)
    r32 = jax.lax.broadcasted_iota(jnp.int32, (32, 32), 0)
    c32 = jax.lax.broadcasted_iota(jnp.int32, (32, 32), 1)
    stril = (c32 < r32).astype(jnp.float32)
    pos = jnp.zeros((32, 128), jnp.float32)
    off_ref[0] = 0
    offs = jnp.float32(0.0)
    for e in range(E):
        m = (ev == e).astype(jnp.float32)
        incl = jnp.dot(m, incl_mat, preferred_element_type=jnp.float32)
        excl = incl - m
        row_tot = incl[:, 127:128]
        rowpref = jnp.dot(stril, row_tot, preferred_element_type=jnp.float32)
        pos = jnp.where(ev == e, offs + rowpref + excl, pos)
        offs = offs + jnp.sum(row_tot)
        off_ref[e + 1] = offs.astype(jnp.int32)
    pos_ref[...] = pos.astype(jnp.int32)

    widx = jnp.int32(0)
    for e in range(E):
        start = off_ref[e]
        end = off_ref[e + 1]
        ft = start // TM
        lt = jnp.where(end > start, (end + TM - 1) // TM, ft)

        def body(t, w, e=e, start=start, end=end):
            eid_ref[w] = e
            mt_ref[w] = t
            lo_ref[w] = jnp.maximum(start - t * TM, 0)
            hi_ref[w] = jnp.minimum(end - t * TM, TM)
            return w + 1
        widx = jax.lax.fori_loop(ft, lt, body, widx)
    last_e = eid_ref[widx - 1]

    def pad(w, _):
        eid_ref[w] = last_e
        mt_ref[w] = NUM_TILES - 1
        lo_ref[w] = 0
        hi_ref[w] = 0
        return 0
    jax.lax.fori_loop(widx, W_ITEMS, pad, 0)
    first_ref[0] = 1
    for i2 in range(1, W_ITEMS):
        first_ref[i2] = jnp.where(mt_ref[i2] != mt_ref[i2 - 1], 1, 0)


def _sort(e2d):
    return pl.pallas_call(
        _sort_body,
        in_specs=[pl.BlockSpec((32, 128), lambda: (0, 0))],
        out_specs=[pl.BlockSpec((32, 128), lambda: (0, 0))]
        + [pl.BlockSpec(memory_space=pltpu.SMEM)] * 5,
        out_shape=[jax.ShapeDtypeStruct((32, 128), jnp.int32)]
        + [jax.ShapeDtypeStruct((W_ITEMS,), jnp.int32)] * 5,
        scratch_shapes=[pltpu.SMEM((E + 1,), jnp.int32)],
    )(e2d)


NCH_D = TPW // CH


def _sc_dispatch(x_flat, pos0, pos1):
    @functools.partial(
        pl.kernel,
        out_type=jax.ShapeDtypeStruct((TK, H), jnp.float32),
        mesh=_sc_mesh(),
        scratch_types=[
            pltpu.VMEM((2, CH, H), jnp.float32),
            pltpu.VMEM((NCH_D, CH), jnp.int32),
            pltpu.VMEM((NCH_D, CH), jnp.int32),
            pltpu.SemaphoreType.DMA,
            pltpu.SemaphoreType.DMA,
            pltpu.SemaphoreType.DMA,
            pltpu.SemaphoreType.DMA,
        ],
    )
    def k(x_hbm, pos0_hbm, pos1_hbm, xs_hbm, rows_v, p0all, p1all,
          semg0, semg1, sem0, sem1):
        wid = lax.axis_index("s") * NC + lax.axis_index("c")
        tbase = wid * TPW
        pltpu.sync_copy(pos0_hbm.at[wid], p0all)
        pltpu.sync_copy(pos1_hbm.at[wid], p1all)
        semg = (semg0, semg1)

        def load(c):
            return pltpu.async_copy(
                x_hbm.at[pl.ds(tbase + c * CH, CH)], rows_v.at[c % 2],
                semg[c % 2])

        pend = load(0)
        prev_sc = None
        for c in range(NCH_D):
            b = c % 2
            if prev_sc is not None:
                prev_sc[0].wait()
                prev_sc[1].wait()
            nl = load(c + 1) if c + 1 < NCH_D else None
            pend.wait()
            i0 = p0all[c]
            i1 = p1all[c]
            s0 = pltpu.async_copy(rows_v.at[b], xs_hbm.at[i0], sem0)
            s1 = pltpu.async_copy(rows_v.at[b], xs_hbm.at[i1], sem1)
            prev_sc = (s0, s1)
            pend = nl
        prev_sc[0].wait()
        prev_sc[1].wait()
    return k(x_flat, pos0.reshape(NW, NCH_D, CH), pos1.reshape(NW, NCH_D, CH))


def _gffn_body(eid_ref, mt_ref, lo_ref, hi_ref, first_ref,
               xs_ref, gw_ref, uw_ref, dw_ref, out_ref):
    i = pl.program_id(0)
    xs = xs_ref[...]
    g = _dotT(xs, gw_ref[0])
    u = _dotT(xs, uw_ref[0])
    h = g * jax.nn.sigmoid(g) * u
    o = _dotT(h, dw_ref[0])
    rows = jax.lax.broadcasted_iota(jnp.int32, (TM, 1), 0)
    mask = ((rows >= lo_ref[i]) & (rows < hi_ref[i])).astype(jnp.float32)
    val = o * mask

    @pl.when(first_ref[i] == 1)
    def _():
        out_ref[...] = val

    @pl.when(first_ref[i] == 0)
    def _():
        out_ref[...] += val


def _grouped_ffn(xs_sorted, egw, euw, edw, eid, mt, lo, hi, first):
    grid_spec = pltpu.PrefetchScalarGridSpec(
        num_scalar_prefetch=5,
        grid=(W_ITEMS,),
        in_specs=[
            pl.BlockSpec((TM, H), lambda i, eid, mt, *p: (mt[i], 0)),
            pl.BlockSpec((1, I, H), lambda i, eid, *p: (eid[i], 0, 0)),
            pl.BlockSpec((1, I, H), lambda i, eid, *p: (eid[i], 0, 0)),
            pl.BlockSpec((1, H, I), lambda i, eid, *p: (eid[i], 0, 0)),
        ],
        out_specs=pl.BlockSpec((TM, H), lambda i, eid, mt, *p: (mt[i], 0)),
    )
    return pl.pallas_call(
        _gffn_body,
        grid_spec=grid_spec,
        out_shape=jax.ShapeDtypeStruct((TK, H), jnp.float32),
    )(eid, mt, lo, hi, first, xs_sorted, egw, euw, edw)


CHC = 8
NCH_C = TPW // CHC


def _sc_combine(shared, routed, pos0, pos1, w0, w1):
    @functools.partial(
        pl.kernel,
        out_type=jax.ShapeDtypeStruct((T, H), jnp.float32),
        mesh=_sc_mesh(),
        scratch_types=[
            pltpu.VMEM((2, CHC, H), jnp.float32),
            pltpu.VMEM((2, CHC, H), jnp.float32),
            pltpu.VMEM((2, CHC, H), jnp.float32),
            pltpu.VMEM((NCH_C, CHC), jnp.int32),
            pltpu.VMEM((NCH_C, CHC), jnp.int32),
            pltpu.VMEM((TPW,), jnp.float32),
            pltpu.VMEM((TPW,), jnp.float32),
            pltpu.SemaphoreType.DMA,
            pltpu.SemaphoreType.DMA,
            pltpu.SemaphoreType.DMA,
            pltpu.SemaphoreType.DMA,
            pltpu.SemaphoreType.DMA,
            pltpu.SemaphoreType.DMA,
        ],
    )
    def k(sh_hbm, rt_hbm, pos0_hbm, pos1_hbm, w0_hbm, w1_hbm, out_hbm,
          acc_v, r0_v, r1_v, p0all, p1all, w0all, w1all,
          semsh0, semsh1, sem0a, sem0b, sem1a, sem1b):
        wid = lax.axis_index("s") * NC + lax.axis_index("c")
        tbase = wid * TPW
        pltpu.sync_copy(pos0_hbm.at[wid], p0all)
        pltpu.sync_copy(pos1_hbm.at[wid], p1all)
        pltpu.sync_copy(w0_hbm.at[wid], w0all)
        pltpu.sync_copy(w1_hbm.at[wid], w1all)
        semsh = (semsh0, semsh1)
        sem0 = (sem0a, sem0b)
        sem1 = (sem1a, sem1b)

        def issue(c):
            b = c % 2
            dsh = pltpu.async_copy(
                sh_hbm.at[pl.ds(tbase + c * CHC, CHC)], acc_v.at[b], semsh[b])
            d0 = pltpu.async_copy(rt_hbm.at[p0all.at[c]], r0_v.at[b], sem0[b])
            d1 = pltpu.async_copy(rt_hbm.at[p1all.at[c]], r1_v.at[b], sem1[b])
            return dsh, d0, d1

        pend = issue(0)
        for c in range(NCH_C):
            b = c % 2
            nxt = issue(c + 1) if c + 1 < NCH_C else None
            for d in pend:
                d.wait()
            gbase = c * CHC
            w0g = w0all[pl.ds((gbase // L) * L, L)]
            w1g = w1all[pl.ds((gbase // L) * L, L)]
            for r in range(CHC):
                lane = jnp.full((L,), (gbase + r) % L, jnp.int32)
                w0s = _splat(w0g, lane)
                w1s = _splat(w1g, lane)

                def body(kk, _):
                    sl = pl.ds(kk * L, L)
                    acc_v[b, r, sl] = (acc_v[b, r, sl]
                                       + w0s * r0_v[b, r, sl]
                                       + w1s * r1_v[b, r, sl])
                    return 0
                lax.fori_loop(0, H // L, body, 0, unroll=8)
            pltpu.sync_copy(acc_v.at[b], out_hbm.at[pl.ds(tbase + gbase, CHC)])
            pend = nxt
    return k(shared, routed, pos0.reshape(NW, NCH_C, CHC),
             pos1.reshape(NW, NCH_C, CHC),
             w0.reshape(NW, TPW), w1.reshape(NW, TPW))


def kernel(x, gate_w, shared_gate_w, shared_up_w, shared_down_w,
           exp_gate_w, exp_up_w, exp_down_w):
    b, s, h = x.shape
    x_flat = x.reshape(-1, h)

    shared, idxT, wT = _shared_router(x_flat, gate_w, shared_gate_w,
                                      shared_up_w, shared_down_w)

    e2d = idxT.reshape(32, 128)
    pos, eid, mt, lo, hi, first = _sort(e2d)
    pos_flat = pos.reshape(TK)
    pos0 = pos_flat[:T]
    pos1 = pos_flat[T:]
    w0 = wT[0]
    w1 = wT[1]

    xs_sorted = _sc_dispatch(x_flat, pos0, pos1)

    routed = _grouped_ffn(xs_sorted, exp_gate_w, exp_up_w, exp_down_w,
                          eid, mt, lo, hi, first)

    out = _sc_combine(shared, routed, pos0, pos1, w0, w1)
    return out.reshape(b, s, h)
